# speedup vs baseline: 1.1478x; 1.1478x over previous
import functools

import numpy as np
import jax
import jax.numpy as jnp
from jax import lax
from jax.experimental import pallas as pl
from jax.experimental.pallas import tpu as pltpu
from jax.experimental.pallas import tpu_sc as plsc

EMBED = 128
WINDOW = 2048
BATCH = 4
TOTAL = BATCH * WINDOW
SCALE = float(np.sqrt(np.float32(EMBED)))

NC = 2
NS = 16
NW = NC * NS
BPW = TOTAL // NW
LANES = 16


def _pos_encoding() -> np.ndarray:
    half = EMBED // 2
    positions = np.arange(WINDOW, dtype=np.float32)[:, None]
    depths = np.arange(half, dtype=np.float32)[None, :] / np.float32(half)
    angle_rates = 1.0 / (10000.0 ** depths)
    angle_rads = positions * angle_rates
    return np.concatenate(
        [np.sin(angle_rads), np.cos(angle_rads)], axis=-1
    ).astype(np.float32)


_POS = _pos_encoding()

_mesh = plsc.VectorSubcoreMesh(core_axis_name="c", subcore_axis_name="s")


@functools.partial(
    pl.kernel,
    mesh=_mesh,
    out_type=jax.ShapeDtypeStruct((TOTAL, EMBED), jnp.float32),
    scratch_types=[
        pltpu.VMEM((BPW,), jnp.int32),
        pltpu.VMEM((BPW, EMBED), jnp.float32),
        pltpu.VMEM((BPW, EMBED), jnp.float32),
        pltpu.SemaphoreType.DMA,
    ],
)
def _emb_kernel(idx_hbm, table_hbm, pos_hbm, out_hbm, idx_v, rows_v, pos_v, sem):
    wid = lax.axis_index("s") * NC + lax.axis_index("c")
    base = wid * BPW
    pltpu.sync_copy(idx_hbm.at[pl.ds(base, BPW)], idx_v)
    gather = pltpu.async_copy(table_hbm.at[idx_v], rows_v, sem)
    pos_base = lax.rem(base, WINDOW)
    pltpu.sync_copy(pos_hbm.at[pl.ds(pos_base, BPW)], pos_v)
    gather.wait()

    def row_step(j, carry):
        for k in range(EMBED // LANES):
            sl = pl.ds(k * LANES, LANES)
            rows_v[j, sl] = rows_v[j, sl] * SCALE + pos_v[j, sl]
        return carry

    lax.fori_loop(0, BPW, row_step, 0)

    pltpu.sync_copy(rows_v, out_hbm.at[pl.ds(base, BPW)])


def kernel(x, table):
    idx = x.reshape(TOTAL).astype(jnp.int32)
    pos = jnp.asarray(_POS)
    out = _emb_kernel(idx, table, pos)
    return out.reshape(BATCH, WINDOW, EMBED)


# speedup vs baseline: 1.1711x; 1.0203x over previous
import functools

import numpy as np
import jax
import jax.numpy as jnp
from jax import lax
from jax.experimental import pallas as pl
from jax.experimental.pallas import tpu as pltpu
from jax.experimental.pallas import tpu_sc as plsc

EMBED = 128
WINDOW = 2048
BATCH = 4
TOTAL = BATCH * WINDOW
SCALE = float(np.sqrt(np.float32(EMBED)))

NC = 2
NS = 16
NW = NC * NS
BPW = TOTAL // NW
LANES = 16
NCHUNK = 4
CH = BPW // NCHUNK


def _pos_encoding() -> np.ndarray:
    half = EMBED // 2
    positions = np.arange(WINDOW, dtype=np.float32)[:, None]
    depths = np.arange(half, dtype=np.float32)[None, :] / np.float32(half)
    angle_rates = 1.0 / (10000.0 ** depths)
    angle_rads = positions * angle_rates
    pos = np.concatenate([np.sin(angle_rads), np.cos(angle_rads)], axis=-1)
    return pos.astype(np.float32).reshape(-1)


_POS = _pos_encoding()

_mesh = plsc.VectorSubcoreMesh(core_axis_name="c", subcore_axis_name="s")


@functools.partial(
    pl.kernel,
    mesh=_mesh,
    out_type=jax.ShapeDtypeStruct((TOTAL, EMBED), jnp.float32),
    scratch_types=[
        pltpu.VMEM((BPW,), jnp.int32),
        pltpu.VMEM((BPW, EMBED), jnp.float32),
        pltpu.VMEM((BPW * EMBED,), jnp.float32),
        pltpu.SemaphoreType.DMA,
    ]
    + [pltpu.SemaphoreType.DMA] * NCHUNK
    + [pltpu.SemaphoreType.DMA] * NCHUNK,
)
def _emb_kernel(idx_hbm, table_hbm, pos_hbm, out_hbm, idx_v, rows_v, pos_v,
                sem_p, *sems):
    gsems = sems[:NCHUNK]
    wsems = sems[NCHUNK:]
    wid = lax.axis_index("s") * NC + lax.axis_index("c")
    base = wid * BPW
    pltpu.sync_copy(idx_hbm.at[pl.ds(base, BPW)], idx_v)
    pos_base = lax.rem(base, WINDOW) * EMBED
    pos_cp = pltpu.async_copy(
        pos_hbm.at[pl.ds(pos_base, BPW * EMBED)], pos_v, sem_p)
    gcps = [
        pltpu.async_copy(
            table_hbm.at[idx_v.at[pl.ds(c * CH, CH)]],
            rows_v.at[pl.ds(c * CH, CH)],
            gsems[c])
        for c in range(NCHUNK)
    ]
    pos_cp.wait()

    wcps = []
    for c in range(NCHUNK):
        gcps[c].wait()

        def row_step(j, carry, _c=c):
            r = _c * CH + j
            pbase = r * EMBED
            for k in range(EMBED // LANES):
                sl = pl.ds(k * LANES, LANES)
                rows_v[r, sl] = (rows_v[r, sl] * SCALE
                                 + pos_v[pl.ds(pbase + k * LANES, LANES)])
            return carry

        lax.fori_loop(0, CH, row_step, 0)
        wcps.append(pltpu.async_copy(
            rows_v.at[pl.ds(c * CH, CH)],
            out_hbm.at[pl.ds(base + c * CH, CH)],
            wsems[c]))
    for w in wcps:
        w.wait()


def kernel(x, table):
    idx = x.reshape(TOTAL).astype(jnp.int32)
    pos = jnp.asarray(_POS)
    out = _emb_kernel(idx, table, pos)
    return out.reshape(BATCH, WINDOW, EMBED)
